# baseline (speedup 1.0000x reference)
import functools

import jax
import jax.numpy as jnp
from jax import lax
from jax.experimental import pallas as pl
from jax.experimental.pallas import tpu as pltpu
from jax.experimental.pallas import tpu_sc as plsc

_NC = 2
_NS = 16
_NW = _NC * _NS
_BL = 128


@functools.lru_cache(maxsize=None)
def _build(b_sz: int, hist: int, vocab: int, d: int, nbuf: int):
    assert b_sz == _NW * _BL and d % 8 == 0
    n_dr = d // 8
    assert hist % nbuf == 0 and hist // nbuf >= 2

    mesh = plsc.VectorSubcoreMesh(
        core_axis_name="c", subcore_axis_name="s",
        num_cores=_NC, num_subcores=_NS)

    @functools.partial(
        pl.kernel,
        out_type=jax.ShapeDtypeStruct((hist, n_dr, _NW, 8, _BL), jnp.float32),
        mesh=mesh,
        scratch_types=[
            pltpu.VMEM((hist, _BL), jnp.int32),
            pltpu.VMEM((nbuf, _BL, d), jnp.float32),
            pltpu.VMEM((nbuf, n_dr, 8, _BL), jnp.float32),
            pltpu.SemaphoreType.DMA((nbuf,)),
            pltpu.SemaphoreType.DMA((nbuf,)),
        ],
        compiler_params=pltpu.CompilerParams(
            use_tc_tiling_on_sc=False, needs_layout_passes=False),
    )
    def k(idxt_hbm, table_hbm, out_hbm, idx_v, g_v, t_v, gsem, ssem):
        wid = lax.axis_index("s") * _NC + lax.axis_index("c")

        pltpu.sync_copy(idxt_hbm.at[:, pl.ds(wid * _BL, _BL)], idx_v)

        lanes = jnp.arange(16, dtype=jnp.int32)
        rows = [lanes + 16 * j for j in range(_BL // 16)]

        def gfire(h, b):
            pltpu.async_copy(
                table_hbm.at[idx_v.at[h]], g_v.at[b], gsem.at[b])

        def gdrain(b):
            pltpu.make_async_copy(
                table_hbm.at[pl.ds(0, _BL)], g_v.at[b], gsem.at[b]).wait()

        def transpose(b):
            @pl.loop(0, d)
            def _dd(dd):
                dr = dd // 8
                ds_ = dd % 8
                col = jnp.full((16,), dd, dtype=jnp.int32)
                for j in range(_BL // 16):
                    v = plsc.load_gather(g_v.at[b], [rows[j], col])
                    t_v.at[b, dr, ds_][pl.ds(16 * j, 16)] = v

        def sfire(h, b):
            pltpu.async_copy(
                t_v.at[b], out_hbm.at[h, :, wid], ssem.at[b])

        def sdrain(b):
            pltpu.make_async_copy(
                t_v.at[b], out_hbm.at[0, :, 0], ssem.at[b]).wait()

        for b in range(nbuf):
            gfire(b, b)
        for b in range(nbuf):
            gdrain(b)
            transpose(b)
            sfire(b, b)
            gfire(nbuf + b, b)

        @pl.loop(nbuf, hist - nbuf, step=nbuf)
        def _steady(h0):
            for b in range(nbuf):
                gdrain(b)
                sdrain(b)
                transpose(b)
                sfire(h0 + b, b)
                gfire(h0 + nbuf + b, b)

        for b in range(nbuf):
            gdrain(b)
            sdrain(b)
            transpose(b)
            sfire(hist - nbuf + b, b)
        for b in range(nbuf):
            sdrain(b)

    return k


def kernel(input_seqs, table):
    batch, hist = input_seqs.shape
    vocab, d = table.shape
    idxt = input_seqs.astype(jnp.int32).T
    o5 = _build(batch, hist, vocab, d, 2)(idxt, table)
    return o5.transpose(2, 4, 0, 1, 3).reshape(batch, hist, d)


# speedup vs baseline: 1.1114x; 1.1114x over previous
import functools

import jax
import jax.numpy as jnp
from jax import lax
from jax.experimental import pallas as pl
from jax.experimental.pallas import tpu as pltpu
from jax.experimental.pallas import tpu_sc as plsc

_NC = 2
_NS = 16
_NW = _NC * _NS
_BL = 128


@functools.lru_cache(maxsize=None)
def _build(b_sz: int, hist: int, vocab: int, d: int, nbuf: int):
    assert b_sz == _NW * _BL and d % 8 == 0
    n_dr = d // 8
    assert hist % nbuf == 0 and hist // nbuf >= 2

    mesh = plsc.VectorSubcoreMesh(
        core_axis_name="c", subcore_axis_name="s",
        num_cores=_NC, num_subcores=_NS)

    @functools.partial(
        pl.kernel,
        out_type=jax.ShapeDtypeStruct((hist, n_dr, _NW, 8, _BL), jnp.float32),
        mesh=mesh,
        scratch_types=[
            pltpu.VMEM((hist, _BL), jnp.int32),
            pltpu.VMEM((nbuf, _BL, d), jnp.float32),
            pltpu.VMEM((nbuf, n_dr, 8, _BL), jnp.float32),
            pltpu.SemaphoreType.DMA((nbuf,)),
            pltpu.SemaphoreType.DMA((nbuf,)),
        ],
        compiler_params=pltpu.CompilerParams(
            use_tc_tiling_on_sc=False, needs_layout_passes=False),
    )
    def k(idxt_hbm, table_hbm, out_hbm, idx_v, g_v, t_v, gsem, ssem):
        wid = lax.axis_index("s") * _NC + lax.axis_index("c")

        pltpu.sync_copy(idxt_hbm.at[:, pl.ds(wid * _BL, _BL)], idx_v)

        lanes = jnp.arange(16, dtype=jnp.int32)
        rows = [lanes + 16 * j for j in range(_BL // 16)]

        def gfire(h, b):
            pltpu.async_copy(
                table_hbm.at[idx_v.at[h]],
                g_v.at[b], gsem.at[b])

        def gdrain(b):
            pltpu.make_async_copy(
                table_hbm.at[pl.ds(0, _BL)],
                g_v.at[b], gsem.at[b]).wait()

        def transpose(b):
            @pl.loop(0, d, unroll=4)
            def _dd(dd):
                dr = dd // 8
                ds_ = dd % 8
                col = jnp.full((16,), dd, dtype=jnp.int32)
                vs = [plsc.load_gather(g_v.at[b], [r, col]) for r in rows]
                for j, v in enumerate(vs):
                    t_v.at[b, dr, ds_][pl.ds(16 * j, 16)] = v

        def sfire(h, b):
            pltpu.async_copy(
                t_v.at[b], out_hbm.at[h, :, wid], ssem.at[b])

        def sdrain(b):
            pltpu.make_async_copy(
                t_v.at[b], out_hbm.at[0, :, 0], ssem.at[b]).wait()

        for b in range(nbuf):
            gfire(b, b)
        for b in range(nbuf):
            gdrain(b)
            transpose(b)
            sfire(b, b)
            gfire(nbuf + b, b)

        @pl.loop(nbuf, hist - nbuf, step=nbuf)
        def _steady(h0):
            for b in range(nbuf):
                gdrain(b)
                sdrain(b)
                transpose(b)
                sfire(h0 + b, b)
                gfire(h0 + nbuf + b, b)

        for b in range(nbuf):
            gdrain(b)
            sdrain(b)
            transpose(b)
            sfire(hist - nbuf + b, b)
        for b in range(nbuf):
            sdrain(b)

    return k


def kernel(input_seqs, table):
    batch, hist = input_seqs.shape
    vocab, d = table.shape
    idxt = input_seqs.astype(jnp.int32).T
    o5 = _build(batch, hist, vocab, d, 2)(idxt, table)
    return o5.transpose(2, 4, 0, 1, 3).reshape(batch, hist, d)
